# baseline (speedup 1.0000x reference)
import jax
import jax.numpy as jnp
from jax.experimental import pallas as pl
from jax.experimental.pallas import tpu as pltpu

MAP_H, MAP_W = 128, 64
NC = 8


def _som_kernel(x_ref, w_ref, out_ref, s_ref, m_ref, d_ref):
    i = pl.program_id(0)
    bkc = w_ref.shape[0]

    @pl.when(i < NC)
    def _scores():
        x = x_ref[...]
        w = w_ref[...]
        xw = jax.lax.dot_general(
            x, w, (((1,), (1,)), ((), ())), preferred_element_type=jnp.float32
        )
        ones_d = jnp.ones((1, x.shape[1]), dtype=jnp.float32)
        wn2 = jax.lax.dot_general(
            ones_d, w * w, (((1,), (1,)), ((), ())),
            preferred_element_type=jnp.float32,
        )
        xn2 = jnp.sum(x * x, axis=1, keepdims=True)
        d2 = jnp.maximum(wn2 + xn2 - 2.0 * xw, 0.0)
        s = -10.0 * jnp.sqrt(d2)
        s_ref[:, pl.ds(i * bkc, bkc)] = s
        m_blk = jnp.max(s, axis=1, keepdims=True)
        e_blk = jnp.sum(jnp.exp(s - m_blk), axis=1, keepdims=True)

        @pl.when(i == 0)
        def _init():
            m_ref[...] = jnp.broadcast_to(m_blk, m_ref.shape)
            d_ref[...] = jnp.broadcast_to(e_blk, d_ref.shape)

        @pl.when(i > 0)
        def _update():
            m_old = m_ref[...]
            m_new = jnp.maximum(m_old, m_blk)
            d_ref[...] = (d_ref[...] * jnp.exp(m_old - m_new)
                          + e_blk * jnp.exp(m_blk - m_new))
            m_ref[...] = m_new

    @pl.when(i == NC)
    def _normalize():
        m = m_ref[:, 0:1]
        inv = 1.0 / d_ref[:, 0:1]
        out_ref[...] = jnp.exp(s_ref[...] - m) * inv


def kernel(input_signal, weight_matrix):
    b, d = input_signal.shape
    kk = weight_matrix.shape[0]
    bkc = kk // NC
    out = pl.pallas_call(
        _som_kernel,
        grid=(NC + 1,),
        in_specs=[
            pl.BlockSpec((b, d), lambda i: (0, 0)),
            pl.BlockSpec((bkc, d), lambda i: (jnp.minimum(i, NC - 1), 0)),
        ],
        out_specs=pl.BlockSpec((b, kk), lambda i: (0, 0)),
        out_shape=jax.ShapeDtypeStruct((b, kk), jnp.float32),
        scratch_shapes=[
            pltpu.VMEM((b, kk), jnp.float32),
            pltpu.VMEM((b, 128), jnp.float32),
            pltpu.VMEM((b, 128), jnp.float32),
        ],
    )(input_signal, weight_matrix)
    return out.reshape(b, MAP_H, MAP_W)


# speedup vs baseline: 1.2017x; 1.2017x over previous
import jax
import jax.numpy as jnp
from jax.experimental import pallas as pl
from jax.experimental.pallas import tpu as pltpu

MAP_H, MAP_W = 128, 64
NC = 8


def _som_kernel(x_ref, w_hbm, out_ref, wv_ref, s_ref, sems):
    bkc = wv_ref.shape[1]
    copies = [
        pltpu.make_async_copy(
            w_hbm.at[pl.ds(i * bkc, bkc), :], wv_ref.at[i], sems.at[i]
        )
        for i in range(NC)
    ]
    for c in copies:
        c.start()
    x = x_ref[...]
    xn2 = jnp.sum(x * x, axis=1, keepdims=True)
    ones_d = jnp.ones((1, x.shape[1]), dtype=jnp.float32)
    for i in range(NC):
        copies[i].wait()
        w = wv_ref[i]
        xw = jax.lax.dot_general(
            x, w, (((1,), (1,)), ((), ())), preferred_element_type=jnp.float32
        )
        wn2 = jax.lax.dot_general(
            ones_d, w * w, (((1,), (1,)), ((), ())),
            preferred_element_type=jnp.float32,
        )
        d2 = jnp.maximum(wn2 + xn2 - 2.0 * xw, 0.0)
        s_ref[:, i * bkc:(i + 1) * bkc] = -10.0 * jnp.sqrt(d2)
    s = s_ref[...]
    m = jnp.max(s, axis=1, keepdims=True)
    e = jnp.exp(s - m)
    out_ref[...] = e / jnp.sum(e, axis=1, keepdims=True)


def kernel(input_signal, weight_matrix):
    b, d = input_signal.shape
    kk = weight_matrix.shape[0]
    bkc = kk // NC
    out = pl.pallas_call(
        _som_kernel,
        in_specs=[
            pl.BlockSpec((b, d), lambda: (0, 0)),
            pl.BlockSpec(memory_space=pltpu.MemorySpace.HBM),
        ],
        out_specs=pl.BlockSpec((b, kk), lambda: (0, 0)),
        out_shape=jax.ShapeDtypeStruct((b, kk), jnp.float32),
        scratch_shapes=[
            pltpu.VMEM((NC, bkc, d), jnp.float32),
            pltpu.VMEM((b, kk), jnp.float32),
            pltpu.SemaphoreType.DMA((NC,)),
        ],
    )(input_signal, weight_matrix)
    return out.reshape(b, MAP_H, MAP_W)
